# baseline (speedup 1.0000x reference)
import functools
import math

import jax
import jax.numpy as jnp
from jax import lax
from jax.experimental import pallas as pl
from jax.experimental.pallas import tpu as pltpu
from jax.experimental.pallas import tpu_sc as plsc


@functools.lru_cache(maxsize=None)
def _build(B, T, D, V):
    info = plsc.get_sparse_core_info()
    NC, NS, L = info.num_cores, info.num_subcores, info.num_lanes
    NW = NC * NS
    NB = 128
    TG = 8
    assert B % (NW * NB) == 0 and T % TG == 0 and D % L == 0
    bpw = B // NW
    nblk = bpw // NB
    ntg = T // TG
    scale = float(math.sqrt(D))

    mesh = plsc.VectorSubcoreMesh(core_axis_name="c", subcore_axis_name="s")

    @functools.partial(
        pl.kernel,
        out_type=jax.ShapeDtypeStruct((B, T, D), jnp.float32),
        mesh=mesh,
        compiler_params=pltpu.CompilerParams(use_tc_tiling_on_sc=False,
                                             needs_layout_passes=False),
        scratch_types=[
            pltpu.VMEM((NB, T), jnp.int32),
            pltpu.VMEM((T, D), jnp.float32),
            pltpu.VMEM((TG, NB), jnp.int32),
            pltpu.VMEM((TG, NB, D), jnp.float32),
            pltpu.SemaphoreType.DMA,
            pltpu.SemaphoreType.DMA,
        ],
    )
    def launch(x_hbm, tok_hbm, pos_hbm, out_hbm, xblk, posblk, idxb, rowb,
               gsem, osem):
        wid = lax.axis_index("s") * NC + lax.axis_index("c")
        b0w = wid * bpw
        pltpu.sync_copy(pos_hbm.at[pl.ds(0, T)], posblk)

        for blk in range(nblk):
            b0 = b0w + blk * NB
            pltpu.sync_copy(x_hbm.at[pl.ds(b0, NB)], xblk)

            def tgbody(tg, carry):
                t0 = pl.multiple_of(tg * TG, TG)

                def ibody(tt, c):
                    tv = jnp.full((L,), t0 + tt, jnp.int32)
                    for j in range(NB // L):
                        rows = lax.iota(jnp.int32, L) + (j * L)
                        idxb[tt, pl.ds(j * L, L)] = plsc.load_gather(
                            xblk, [rows, tv])
                    return c

                lax.fori_loop(0, TG, ibody, 0)
                gathers = [
                    pltpu.async_copy(tok_hbm.at[idxb.at[tt]], rowb.at[tt],
                                     gsem)
                    for tt in range(TG)
                ]
                for g in gathers:
                    g.wait()
                for tt in range(TG):
                    pv = [posblk[t0 + tt, pl.ds(k * L, L)]
                          for k in range(D // L)]

                    def rbody(r, c, tt=tt, pv=pv):
                        for k in range(D // L):
                            sl = (tt, r, pl.ds(k * L, L))
                            rowb[sl] = rowb[sl] * scale + pv[k]
                        return c

                    lax.fori_loop(0, NB, rbody, 0, unroll=2)

                def obody(i, c):
                    pltpu.async_copy(rowb.at[:, i, :],
                                     out_hbm.at[b0 + i, pl.ds(t0, TG)], osem)
                    return c

                lax.fori_loop(0, NB, obody, 0)
                pltpu.make_async_copy(
                    out_hbm.at[pl.ds(0, TG), pl.ds(0, NB)], rowb, osem).wait()
                return carry

            lax.fori_loop(0, ntg, tgbody, 0)

    return launch


def kernel(x, token_table, pos_table):
    B, T = x.shape
    V, D = token_table.shape
    launch = _build(B, T, D, V)
    return launch(x.astype(jnp.int32), token_table, pos_table)
